# Initial kernel scaffold; baseline (speedup 1.0000x reference)
import jax
import jax.numpy as jnp
from jax.experimental import pallas as pl


def kernel(n_feat, edge_index, e_feat, lin0_W, lin0_b, en1_W, en1_b, en2_W, en2_b, bn_gamma, bn_beta, yl_W, yl_b, yl2_W, yl2_b):
    raise NotImplementedError("write your pallas kernel here")


# speedup vs baseline: 1.5580x; 1.5580x over previous
import functools

import jax
import jax.numpy as jnp
from jax import lax
from jax.experimental import pallas as pl
from jax.experimental.pallas import tpu as pltpu
from jax.experimental.pallas import tpu_sc as plsc

N = 10000
E = 160000
DIN = 128
H = 32
ED = 16
EH = 128
STEPS = 3
EPS = 1e-5

NC, NS = 2, 16
NW = NC * NS
CH = 128
EP = 163840
CHN = EP // CH
RPW = CHN // NW
RPG = 4
NGRP = RPW // RPG
EPG = RPG * CH
W = 128
ACCN = 10240
TRASH = N
HALFN = ACCN // NC
ACCR = HALFN + CH
ZR = ACCR // NS
ORT = HALFN // NS
SRPT = CHN // NS
SGRP = SRPT // RPG

NB = 1000
EB = 640


def _lin0_body(x_ref, w_ref, b_ref, o_ref):
    o = jnp.maximum(
        jnp.dot(x_ref[...], w_ref[...], preferred_element_type=jnp.float32)
        + b_ref[...], 0.0)
    o_ref[...] = jnp.concatenate(
        [o, jnp.zeros((o.shape[0], W - H), jnp.float32)], axis=1)


_lin0 = pl.pallas_call(
    _lin0_body,
    grid=(N // NB,),
    in_specs=[pl.BlockSpec((NB, DIN), lambda i: (i, 0)),
              pl.BlockSpec((DIN, H), lambda i: (0, 0)),
              pl.BlockSpec((1, H), lambda i: (0, 0))],
    out_specs=pl.BlockSpec((NB, W), lambda i: (i, 0)),
    out_shape=jax.ShapeDtypeStruct((N, W), jnp.float32),
    name="lin0",
)


def _msg_body(ef_ref, srch_ref, w1_ref, b1_ref, w2_ref, b2_ref, exp_ref,
              o_ref):
    eh = jnp.maximum(
        jnp.dot(ef_ref[...], w1_ref[...], preferred_element_type=jnp.float32)
        + b1_ref[...], 0.0)
    w = jnp.dot(eh.astype(jnp.bfloat16), w2_ref[...],
                preferred_element_type=jnp.float32) + b2_ref[...]
    se = jnp.dot(srch_ref[:, 0:H].astype(jnp.bfloat16), exp_ref[...],
                 preferred_element_type=jnp.float32)
    p = se * w
    s = p[:, 0:128]
    for v in range(1, 8):
        s = s + p[:, v * 128:(v + 1) * 128]
    m = s[:, 0:32] + s[:, 32:64] + s[:, 64:96] + s[:, 96:128]
    o_ref[...] = jnp.concatenate(
        [m, jnp.zeros((m.shape[0], W - H), jnp.float32)], axis=1)


_msg = pl.pallas_call(
    _msg_body,
    grid=(EP // EB,),
    in_specs=[pl.BlockSpec((EB, ED), lambda i: (i, 0)),
              pl.BlockSpec((EB, W), lambda i: (i, 0)),
              pl.BlockSpec((ED, EH), lambda i: (0, 0)),
              pl.BlockSpec((1, EH), lambda i: (0, 0)),
              pl.BlockSpec((EH, H * H), lambda i: (0, 0)),
              pl.BlockSpec((1, H * H), lambda i: (0, 0)),
              pl.BlockSpec((H, H * H), lambda i: (0, 0))],
    out_specs=pl.BlockSpec((EB, W), lambda i: (i, 0)),
    out_shape=jax.ShapeDtypeStruct((EP, W), jnp.float32),
    name="message",
)


def _addrelu_body(a_ref, o_ref):
    o_ref[...] = jnp.maximum(a_ref[...], 0.0)


_addrelu = pl.pallas_call(
    _addrelu_body,
    grid=(N // NB,),
    in_specs=[pl.BlockSpec((NB, W), lambda i: (i, 0))],
    out_specs=pl.BlockSpec((NB, W), lambda i: (i, 0)),
    out_shape=jax.ShapeDtypeStruct((N, W), jnp.float32),
    name="relu",
)


def _stats_body(a_ref, o_ref, s1_ref, s2_ref):
    i = pl.program_id(0)
    o = jnp.maximum(a_ref[:, 0:H], 0.0)
    o_ref[...] = o

    @pl.when(i == 0)
    def _():
        s1_ref[...] = jnp.zeros_like(s1_ref)
        s2_ref[...] = jnp.zeros_like(s2_ref)

    s1_ref[...] += jnp.sum(o, axis=0, keepdims=True)
    s2_ref[...] += jnp.sum(o * o, axis=0, keepdims=True)


_stats = pl.pallas_call(
    _stats_body,
    grid=(N // NB,),
    in_specs=[pl.BlockSpec((NB, W), lambda i: (i, 0))],
    out_specs=[pl.BlockSpec((NB, H), lambda i: (i, 0)),
               pl.BlockSpec((1, H), lambda i: (0, 0)),
               pl.BlockSpec((1, H), lambda i: (0, 0))],
    out_shape=[jax.ShapeDtypeStruct((N, H), jnp.float32),
               jax.ShapeDtypeStruct((1, H), jnp.float32),
               jax.ShapeDtypeStruct((1, H), jnp.float32)],
    name="final_relu_stats",
)


def _heads_body(o_ref, s1_ref, s2_ref, g_ref, be_ref, w1_ref, b1_ref,
                w2_ref, b2_ref, y1_ref, y2_ref):
    mean = s1_ref[...] / N
    var = s2_ref[...] / N - mean * mean
    inv = lax.rsqrt(var + EPS) * g_ref[...]
    yb = (o_ref[...] - mean) * inv + be_ref[...]
    z1 = jnp.dot(yb, w1_ref[...], preferred_element_type=jnp.float32) + b1_ref[...]
    z2 = jnp.dot(yb, w2_ref[...], preferred_element_type=jnp.float32) + b2_ref[...]
    y1_ref[...] = 1.0 / (1.0 + jnp.exp(-z1))
    y2_ref[...] = 1.0 / (1.0 + jnp.exp(-z2))


_heads = pl.pallas_call(
    _heads_body,
    grid=(N // NB,),
    in_specs=[pl.BlockSpec((NB, H), lambda i: (i, 0)),
              pl.BlockSpec((1, H), lambda i: (0, 0)),
              pl.BlockSpec((1, H), lambda i: (0, 0)),
              pl.BlockSpec((1, H), lambda i: (0, 0)),
              pl.BlockSpec((1, H), lambda i: (0, 0)),
              pl.BlockSpec((H, 2), lambda i: (0, 0)),
              pl.BlockSpec((1, 2), lambda i: (0, 0)),
              pl.BlockSpec((H, 1), lambda i: (0, 0)),
              pl.BlockSpec((1, 1), lambda i: (0, 0))],
    out_specs=[pl.BlockSpec((NB, 2), lambda i: (i, 0)),
               pl.BlockSpec((NB, 1), lambda i: (i, 0))],
    out_shape=[jax.ShapeDtypeStruct((N, 2), jnp.float32),
               jax.ShapeDtypeStruct((N, 1), jnp.float32)],
    name="bn_heads",
)


@functools.lru_cache(maxsize=1)
def _sc_kernels():
    mesh = plsc.VectorSubcoreMesh(core_axis_name="c", subcore_axis_name="s",
                                  num_cores=NC, num_subcores=NS)

    @functools.partial(
        pl.kernel,
        out_type=jax.ShapeDtypeStruct((EP, W), jnp.float32),
        mesh=mesh,
        scratch_types=[pltpu.VMEM((RPG, CH), jnp.int32),
                       pltpu.VMEM((EPG, W), jnp.float32),
                       pltpu.SemaphoreType.DMA],
        name="sc_gather",
    )
    def gather_sc(table_hbm, idx_hbm, out_hbm, idxd, rows, sem):
        c = lax.axis_index("c")
        s = lax.axis_index("s")
        wid = s * NC + c

        def grp(g, carry):
            r0 = wid * RPW + g * RPG
            pltpu.sync_copy(idx_hbm.at[pl.ds(r0, RPG)], idxd)
            cps = [pltpu.async_copy(table_hbm.at[idxd.at[j]],
                                    rows.at[pl.ds(j * CH, CH)], sem)
                   for j in range(RPG)]
            for cp in cps:
                cp.wait()
            pltpu.sync_copy(rows, out_hbm.at[pl.ds(r0 * CH, EPG)])
            return carry

        lax.fori_loop(0, NGRP, grp, 0)

    @functools.partial(
        pl.kernel,
        out_type=jax.ShapeDtypeStruct((ACCN, W), jnp.float32),
        mesh=mesh,
        scratch_types=[pltpu.VMEM((RPG, CH), jnp.int32),
                       pltpu.VMEM((EPG, W), jnp.float32),
                       pltpu.VMEM_SHARED((ACCR, W), jnp.float32),
                       pltpu.SemaphoreType.DMA],
        name="sc_scatter_add",
    )
    def scatter_sc(msg_hbm, idx_hbm, z_hbm, out_hbm, idxd, rows, acc, sem):
        c = lax.axis_index("c")
        s = lax.axis_index("s")
        base = c * HALFN

        pltpu.sync_copy(z_hbm, acc.at[pl.ds(s * ZR, ZR)])
        plsc.subcore_barrier()

        def grp(g, carry):
            r0 = s * SRPT + g * RPG
            pltpu.sync_copy(idx_hbm.at[pl.ds(r0, RPG)], idxd)
            pltpu.sync_copy(msg_hbm.at[pl.ds(r0 * CH, EPG)], rows)
            for j in range(RPG):
                for k in range(CH // 16):
                    v = idxd[j, pl.ds(k * 16, 16)]
                    lv = v - base
                    oob = jnp.logical_or(lv < 0, lv >= HALFN)
                    idxd[j, pl.ds(k * 16, 16)] = jnp.where(oob, HALFN, lv)
            cps = [pltpu.async_copy(rows.at[pl.ds(j * CH, CH)],
                                    acc.at[idxd.at[j]], sem, add=True)
                   for j in range(RPG)]
            for cp in cps:
                cp.wait()
            return carry

        lax.fori_loop(0, SGRP, grp, 0)
        plsc.subcore_barrier()
        pltpu.sync_copy(acc.at[pl.ds(s * ORT, ORT)],
                        out_hbm.at[pl.ds(base + s * ORT, ORT)])

    return gather_sc, scatter_sc


def kernel(n_feat, edge_index, e_feat, lin0_W, lin0_b, en1_W, en1_b, en2_W,
           en2_b, bn_gamma, bn_beta, yl_W, yl_b, yl2_W, yl2_b):
    src = edge_index[0].astype(jnp.int32)
    dst = edge_index[1].astype(jnp.int32)
    pad = EP - E
    src2 = jnp.concatenate([src, jnp.zeros((pad,), jnp.int32)]).reshape(CHN, CH)
    dst2 = jnp.concatenate([dst, jnp.full((pad,), TRASH, jnp.int32)]).reshape(CHN, CH)
    ef_p = jnp.concatenate([e_feat, jnp.zeros((pad, ED), jnp.float32)], axis=0)
    zrows = jnp.zeros((ZR, W), jnp.float32)

    w2bf = en2_W.astype(jnp.bfloat16)
    expm = jnp.repeat(jnp.eye(H, dtype=jnp.bfloat16), H, axis=1)
    b1r = en1_b.reshape(1, EH)
    b2r = en2_b.reshape(1, H * H)

    gather_sc, scatter_sc = _sc_kernels()
    table = _lin0(n_feat, lin0_W, lin0_b.reshape(1, H))
    agg = None
    for step in range(STEPS):
        srch = gather_sc(table, src2)
        msg = _msg(ef_p, srch, en1_W, b1r, w2bf, b2r, expm)
        agg = scatter_sc(msg, dst2, zrows)
        if step < STEPS - 1:
            table = _addrelu(agg)

    out, s1, s2 = _stats(agg)
    y1, y2 = _heads(out, s1, s2, bn_gamma.reshape(1, H), bn_beta.reshape(1, H),
                    yl_W, yl_b.reshape(1, 2), yl2_W, yl2_b.reshape(1, 1))
    return (y1, y2)
